# baseline (speedup 1.0000x reference)
import jax
import jax.numpy as jnp
from jax import lax
from jax.experimental import pallas as pl
from jax.experimental.pallas import tpu as pltpu
from jax.experimental.pallas import tpu_sc as plsc

B = 16384
F = 26
VOCAB = 100000
D = 32

NC = 2
NS = 16
NW = NC * NS
L = 16
BH = B // 2
NR = BH // 128
GROUPS = (7, 7, 6, 6)


def _make_body(g_fields):
    def _body(idx_hbm, tab_hbm, out_hbm, idx_v, strip_v, stg_v, row_v, sem):
        wid = lax.axis_index("s") * NC + lax.axis_index("c")
        lane = lax.iota(jnp.int32, L)

        def strip(i, carry):
            p = wid * g_fields + i
            f = p // D
            d = p - f * D
            pltpu.sync_copy(tab_hbm.at[f, d], strip_v)
            base = f * 4096 + (d // 8) * 1024 + (d % 8)

            for h in range(2):
                pltpu.sync_copy(idx_hbm.at[f, pl.ds(h * BH, BH)], idx_v)

                def row(bc, carry2):
                    for j in range(8):
                        raw = idx_v[pl.ds(bc * 128 + j * L, L)]
                        v = jnp.clip(raw, 0, VOCAB - 1)
                        stg_v[bc, pl.ds(j * L, L)] = plsc.load_gather(
                            strip_v, [v])
                    return carry2
                lax.fori_loop(0, NR, row, 0)

                for g in range(NR // L):
                    row_v[pl.ds(g * L, L)] = (
                        base + h * 512 + (g * L + lane) * 8)
                pltpu.async_copy(stg_v, out_hbm.at[row_v], sem).wait()
            return carry

        lax.fori_loop(0, g_fields, strip, 0)

    return _body


def kernel(indices, tables):
    idx_t = jnp.swapaxes(indices, 0, 1).astype(jnp.int32)
    tab_t = jnp.swapaxes(tables, 1, 2)
    mesh = plsc.VectorSubcoreMesh(
        core_axis_name="c", subcore_axis_name="s",
        num_cores=NC, num_subcores=NS,
    )
    pieces = []
    f0 = 0
    for gf in GROUPS:
        fn = pl.kernel(
            _make_body(gf),
            out_type=jax.ShapeDtypeStruct((gf * D * B // 128, 128),
                                          jnp.float32),
            mesh=mesh,
            scratch_types=[
                pltpu.VMEM((BH,), jnp.int32),
                pltpu.VMEM((VOCAB,), jnp.float32),
                pltpu.VMEM((NR, 128), jnp.float32),
                pltpu.VMEM((NR,), jnp.int32),
                pltpu.SemaphoreType.DMA,
            ],
            compiler_params=pltpu.CompilerParams(
                use_tc_tiling_on_sc=False, needs_layout_passes=False),
        )
        pieces.append(fn(idx_t[f0:f0 + gf], tab_t[f0:f0 + gf]))
        f0 += gf
    out2d = jnp.concatenate(pieces, axis=0)
    out = jnp.transpose(
        out2d.reshape(F, D // 8, B // 128, 8, 128), (2, 4, 0, 1, 3)
    ).reshape(B, F, D)
    return out


# speedup vs baseline: 2.6749x; 2.6749x over previous
import jax
import jax.numpy as jnp
from jax import lax
from jax.experimental import pallas as pl
from jax.experimental.pallas import tpu as pltpu
from jax.experimental.pallas import tpu_sc as plsc

B = 16384
F = 26
VOCAB = 100000
D = 32

NC = 2
NS = 16
NW = NC * NS
L = 16
NP = F * D
PPT = NP // NW
BH = B // 2
NR = BH // 128
OROWS = B * F * D // 128


def _body(idx_hbm, tab_hbm, out_hbm, idx_v, strip_v, stg_v, row_v, sem):
    wid = lax.axis_index("s") * NC + lax.axis_index("c")
    lane = lax.iota(jnp.int32, L)

    def strip(i, carry):
        p = wid * PPT + i
        f = p // D
        d = p - f * D
        pltpu.sync_copy(tab_hbm.at[f, d], strip_v)
        base = f * 4096 + (d // 8) * 1024 + (d % 8)

        for h in range(2):
            pltpu.sync_copy(idx_hbm.at[f, pl.ds(h * BH, BH)], idx_v)

            def row(bc, carry2):
                for j in range(8):
                    raw = idx_v[pl.ds(bc * 128 + j * L, L)]
                    v = jnp.clip(raw, 0, VOCAB - 1)
                    stg_v[bc, pl.ds(j * L, L)] = plsc.load_gather(
                        strip_v, [v])
                return carry2
            lax.fori_loop(0, NR, row, 0)

            for g in range(NR // L):
                row_v[pl.ds(g * L, L)] = (
                    base + h * 512 + (g * L + lane) * 8)
            pltpu.async_copy(stg_v, out_hbm.at[row_v], sem).wait()
        return carry

    lax.fori_loop(0, PPT, strip, 0)


def kernel(indices, tables):
    idx_t = jnp.swapaxes(indices, 0, 1).astype(jnp.int32)
    tab_t = jnp.swapaxes(tables, 1, 2)
    mesh = plsc.VectorSubcoreMesh(
        core_axis_name="c", subcore_axis_name="s",
        num_cores=NC, num_subcores=NS,
    )
    fn = pl.kernel(
        _body,
        out_type=jax.ShapeDtypeStruct((OROWS, 128), jnp.float32),
        mesh=mesh,
        scratch_types=[
            pltpu.VMEM((BH,), jnp.int32),
            pltpu.VMEM((VOCAB,), jnp.float32),
            pltpu.VMEM((NR, 128), jnp.float32),
            pltpu.VMEM((NR,), jnp.int32),
            pltpu.SemaphoreType.DMA,
        ],
        compiler_params=pltpu.CompilerParams(
            use_tc_tiling_on_sc=True, needs_layout_passes=False),
    )
    out2d = fn(idx_t, tab_t)
    out = jnp.transpose(
        out2d.reshape(F, D // 8, B // 128, 8, 128), (2, 4, 0, 1, 3)
    ).reshape(B, F, D)
    return out


# speedup vs baseline: 3.2051x; 1.1982x over previous
import jax
import jax.numpy as jnp
from jax import lax
from jax.experimental import pallas as pl
from jax.experimental.pallas import tpu as pltpu
from jax.experimental.pallas import tpu_sc as plsc

B = 16384
F = 26
VOCAB = 100000
D = 32

NC = 2
NS = 16
NW = NC * NS
L = 16
NP = F * D
PPT = NP // NW
BH = B // 2
NR = BH // 128
OROWS = B * F * D // 128


def _body(idx_hbm, tab_hbm, out_hbm, idx_v, strip_v, stg_v, row_v, sem):
    wid = lax.axis_index("s") * NC + lax.axis_index("c")
    lane = lax.iota(jnp.int32, L)

    def strip(i, last_f):
        p = wid * PPT + i
        f = p // D
        d = p - f * D
        pltpu.sync_copy(tab_hbm.at[f, d], strip_v)
        base = f * 4096 + (d // 8) * 1024 + (d % 8)

        @pl.when(f != last_f)
        def _():
            pltpu.sync_copy(idx_hbm.at[f], idx_v)

        for h in range(2):
            def row(bc, carry2):
                for j in range(8):
                    raw = idx_v[pl.ds(h * BH + bc * 128 + j * L, L)]
                    v = jnp.clip(raw, 0, VOCAB - 1)
                    stg_v[bc, pl.ds(j * L, L)] = plsc.load_gather(
                        strip_v, [v])
                return carry2
            lax.fori_loop(0, NR, row, 0)

            for g in range(NR // L):
                row_v[pl.ds(g * L, L)] = (
                    base + h * 512 + (g * L + lane) * 8)
            pltpu.async_copy(stg_v, out_hbm.at[row_v], sem).wait()
        return f

    lax.fori_loop(0, PPT, strip, -1)


def kernel(indices, tables):
    idx_t = jnp.swapaxes(indices, 0, 1).astype(jnp.int32)
    tab_t = jnp.swapaxes(tables, 1, 2)
    mesh = plsc.VectorSubcoreMesh(
        core_axis_name="c", subcore_axis_name="s",
        num_cores=NC, num_subcores=NS,
    )
    fn = pl.kernel(
        _body,
        out_type=jax.ShapeDtypeStruct((OROWS, 128), jnp.float32),
        mesh=mesh,
        scratch_types=[
            pltpu.VMEM((B,), jnp.int32),
            pltpu.VMEM((VOCAB,), jnp.float32),
            pltpu.VMEM((NR, 128), jnp.float32),
            pltpu.VMEM((NR,), jnp.int32),
            pltpu.SemaphoreType.DMA,
        ],
        compiler_params=pltpu.CompilerParams(
            use_tc_tiling_on_sc=True, needs_layout_passes=False),
    )
    out2d = fn(idx_t, tab_t)
    out = jnp.transpose(
        out2d.reshape(F, D // 8, B // 128, 8, 128), (2, 4, 0, 1, 3)
    ).reshape(B, F, D)
    return out
